# baseline (speedup 1.0000x reference)
import functools

import jax
import jax.numpy as jnp
from jax import lax
from jax.experimental import pallas as pl
from jax.experimental.pallas import tpu as pltpu
from jax.experimental.pallas import tpu_sc as plsc

SIZE = 100000
HALF = 50000
NEV = 3200000
POOLW = 10
NBUCK = 5000
PSTR = 5120
ROWW = 128
ACC_ROWS = 512
ACC_N = ACC_ROWS * ROWW
NW = 32
EW = NEV // NW
CHUNK = 4000
NCH = EW // CHUNK
GROUPS = CHUNK // 16
NSLOT = 3
NPRIME = 2
RPT = ACC_ROWS // 16
IDX_MINOR = 128
IDX_TILES = ACC_ROWS // IDX_MINOR


def _sc_partials(index, rate, starttime, endtime, tvec):
    mesh = plsc.VectorSubcoreMesh(core_axis_name="c", subcore_axis_name="s")

    @functools.partial(
        pl.kernel,
        mesh=mesh,
        out_type=jax.ShapeDtypeStruct((2, ACC_ROWS, ROWW), jnp.float32),
        scratch_types=(
            [pltpu.VMEM((CHUNK,), jnp.int32)] * NSLOT
            + [pltpu.VMEM((CHUNK,), jnp.float32)] * (3 * NSLOT)
            + [
                pltpu.VMEM((ACC_ROWS, ROWW), jnp.float32),
                pltpu.VMEM((16,), jnp.float32),
                pltpu.VMEM((IDX_TILES, IDX_MINOR), jnp.int32),
                pltpu.VMEM((RPT, ROWW), jnp.float32),
                pltpu.VMEM_SHARED((ACC_ROWS, ROWW), jnp.float32),
            ]
            + [pltpu.SemaphoreType.DMA] * (4 * NSLOT)
        ),
        compiler_params=pltpu.CompilerParams(needs_layout_passes=False),
    )
    def body(idx_h, rate_h, st_h, en_h, tv_h, out_h, *scratch):
        ib = scratch[0:NSLOT]
        rb = scratch[NSLOT:2 * NSLOT]
        sb = scratch[2 * NSLOT:3 * NSLOT]
        eb = scratch[3 * NSLOT:4 * NSLOT]
        acc, tb, rows_i, wb, shacc = scratch[4 * NSLOT:4 * NSLOT + 5]
        sems = scratch[4 * NSLOT + 5:]
        cid = lax.axis_index("c")
        sid = lax.axis_index("s")
        wid = cid * 16 + sid
        base = wid * EW

        def copies(slot, g):
            off = base + g * CHUNK
            return [
                pltpu.make_async_copy(idx_h.at[pl.ds(off, CHUNK)],
                                      ib[slot], sems[slot * 4 + 0]),
                pltpu.make_async_copy(rate_h.at[pl.ds(off, CHUNK)],
                                      rb[slot], sems[slot * 4 + 1]),
                pltpu.make_async_copy(st_h.at[pl.ds(off, CHUNK)],
                                      sb[slot], sems[slot * 4 + 2]),
                pltpu.make_async_copy(en_h.at[pl.ds(off, CHUNK)],
                                      eb[slot], sems[slot * 4 + 3]),
            ]

        def start(slot, g):
            for c in copies(slot, g):
                c.start()

        def wait(slot, g):
            for c in copies(slot, g):
                c.wait()

        for p in range(NPRIME):
            start(p, p)
        pltpu.sync_copy(tv_h, tb)
        tv = tb[...]

        zero16 = jnp.zeros((16,), jnp.float32)

        @plsc.parallel_loop(0, ACC_N // 16, unroll=8)
        def zacc(i):
            acc[i >> 3, pl.ds((i & 7) * 16, 16)] = zero16

        for k in range(IDX_TILES):
            for j in range(IDX_MINOR // 16):
                rows_i[k, pl.ds(j * 16, 16)] = (
                    lax.iota(jnp.int32, 16) + (k * IDX_MINOR + j * 16))

        @plsc.parallel_loop(0, RPT * 8, unroll=8)
        def zwb(i):
            wb[i >> 3, pl.ds((i & 7) * 16, 16)] = zero16

        pltpu.sync_copy(wb, shacc.at[pl.ds(sid * RPT, RPT)])

        def chunk_compute(b):
            @plsc.parallel_loop(0, GROUPS, unroll=10)
            def grp(i):
                idx = ib[b][pl.ds(i * 16, 16)]
                rt = rb[b][pl.ds(i * 16, 16)]
                st = sb[b][pl.ds(i * 16, 16)]
                en = eb[b][pl.ds(i * 16, 16)]
                m = (st <= tv) & (tv < en)
                hm = idx < HALF
                n = jnp.where(hm, idx, idx - HALF)
                q = ((n.astype(jnp.uint32) * jnp.uint32(52429))
                     >> jnp.uint32(19)).astype(jnp.int32)
                off2 = n * PSTR + PSTR - q * (10 * PSTR)
                pos = q + jnp.where(hm, 0, off2)
                plsc.addupdate_scatter(acc, [pos >> 7, pos & 127], rt,
                                       mask=m)

        def outer(it, carry):
            g0 = it * NSLOT
            for b in range(NSLOT):
                g = g0 + b
                wait(b, g)

                @pl.when(g + NPRIME < NCH)
                def _():
                    start((b + NPRIME) % NSLOT, g + NPRIME)

                chunk_compute(b)
            return carry

        lax.fori_loop(0, (NCH - 1) // NSLOT, outer, 0)
        wait((NCH - 1) % NSLOT, NCH - 1)
        chunk_compute((NCH - 1) % NSLOT)

        plsc.subcore_barrier()
        for k in range(IDX_TILES):
            pltpu.sync_copy(acc.at[pl.ds(k * IDX_MINOR, IDX_MINOR)],
                            shacc.at[rows_i.at[k]], add=True)
        plsc.subcore_barrier()
        pltpu.sync_copy(shacc.at[pl.ds(sid * RPT, RPT)], wb)
        pltpu.sync_copy(wb, out_h.at[cid, pl.ds(sid * RPT, RPT)])

    return body(index, rate, starttime, endtime, tvec)


def _tc_combine(partials):
    PR = PSTR // ROWW

    def body(p_ref, out_ref):
        a = p_ref[0] + p_ref[1]
        s = a[0:PR, :].reshape(-1)
        m = a[PR:2 * PR, :]
        for w in range(1, POOLW):
            m = jnp.maximum(m, a[PR * (w + 1):PR * (w + 2), :])
        out_ref[pl.ds(0, NBUCK)] = s[:NBUCK]
        out_ref[pl.ds(NBUCK, NBUCK)] = m.reshape(-1)[:NBUCK]

    return pl.pallas_call(
        body,
        out_shape=jax.ShapeDtypeStruct((2 * NBUCK,), jnp.float32),
    )(partials)


@jax.jit
def kernel(index, rate, starttime, endtime, t):
    tvec = jnp.full((16,), t, jnp.float32)
    parts = _sc_partials(index.astype(jnp.int32), rate, starttime,
                         endtime, tvec)
    return _tc_combine(parts)


# speedup vs baseline: 1.0079x; 1.0079x over previous
import functools

import jax
import jax.numpy as jnp
from jax import lax
from jax.experimental import pallas as pl
from jax.experimental.pallas import tpu as pltpu
from jax.experimental.pallas import tpu_sc as plsc

SIZE = 100000
HALF = 50000
NEV = 3200000
POOLW = 10
NBUCK = 5000
PSTR = 5120
ROWW = 128
ACC_ROWS = 512
ACC_N = ACC_ROWS * ROWW
NW = 32
EW = NEV // NW
CHUNK = 4000
NCH = EW // CHUNK
GROUPS = CHUNK // 16
NSLOT = 3
NPRIME = 2
RPT = ACC_ROWS // 16
IDX_MINOR = 128
IDX_TILES = ACC_ROWS // IDX_MINOR


def _sc_partials(index, rate, starttime, endtime, tvec):
    mesh = plsc.VectorSubcoreMesh(core_axis_name="c", subcore_axis_name="s")

    @functools.partial(
        pl.kernel,
        mesh=mesh,
        out_type=jax.ShapeDtypeStruct((2, ACC_ROWS, ROWW), jnp.float32),
        scratch_types=(
            [pltpu.VMEM((CHUNK,), jnp.int32)] * NSLOT
            + [pltpu.VMEM((CHUNK,), jnp.float32)] * (3 * NSLOT)
            + [
                pltpu.VMEM((ACC_ROWS, ROWW), jnp.float32),
                pltpu.VMEM((16,), jnp.float32),
                pltpu.VMEM((IDX_TILES, IDX_MINOR), jnp.int32),
                pltpu.VMEM((RPT, ROWW), jnp.float32),
                pltpu.VMEM_SHARED((ACC_ROWS, ROWW), jnp.float32),
            ]
            + [pltpu.SemaphoreType.DMA] * (4 * NSLOT)
        ),
        compiler_params=pltpu.CompilerParams(needs_layout_passes=False),
    )
    def body(idx_h, rate_h, st_h, en_h, tv_h, out_h, *scratch):
        ib = scratch[0:NSLOT]
        rb = scratch[NSLOT:2 * NSLOT]
        sb = scratch[2 * NSLOT:3 * NSLOT]
        eb = scratch[3 * NSLOT:4 * NSLOT]
        acc, tb, rows_i, wb, shacc = scratch[4 * NSLOT:4 * NSLOT + 5]
        sems = scratch[4 * NSLOT + 5:]
        cid = lax.axis_index("c")
        sid = lax.axis_index("s")
        wid = cid * 16 + sid
        base = wid * EW

        def copies(slot, g):
            off = base + g * CHUNK
            return [
                pltpu.make_async_copy(idx_h.at[pl.ds(off, CHUNK)],
                                      ib[slot], sems[slot * 4 + 0]),
                pltpu.make_async_copy(rate_h.at[pl.ds(off, CHUNK)],
                                      rb[slot], sems[slot * 4 + 1]),
                pltpu.make_async_copy(st_h.at[pl.ds(off, CHUNK)],
                                      sb[slot], sems[slot * 4 + 2]),
                pltpu.make_async_copy(en_h.at[pl.ds(off, CHUNK)],
                                      eb[slot], sems[slot * 4 + 3]),
            ]

        def start(slot, g):
            for c in copies(slot, g):
                c.start()

        def wait(slot, g):
            for c in copies(slot, g):
                c.wait()

        for p in range(NPRIME):
            start(p, p)
        pltpu.sync_copy(tv_h, tb)
        tv = tb[...]

        zero16 = jnp.zeros((16,), jnp.float32)

        @plsc.parallel_loop(0, ACC_N // 16, unroll=8)
        def zacc(i):
            acc[i >> 3, pl.ds((i & 7) * 16, 16)] = zero16

        for k in range(IDX_TILES):
            for j in range(IDX_MINOR // 16):
                rows_i[k, pl.ds(j * 16, 16)] = (
                    lax.iota(jnp.int32, 16) + (k * IDX_MINOR + j * 16))

        @plsc.parallel_loop(0, RPT * 8, unroll=8)
        def zwb(i):
            wb[i >> 3, pl.ds((i & 7) * 16, 16)] = zero16

        pltpu.sync_copy(wb, shacc.at[pl.ds(sid * RPT, RPT)])

        def chunk_compute(b):
            @plsc.parallel_loop(0, GROUPS, unroll=5)
            def grp(i):
                idx = ib[b][pl.ds(i * 16, 16)]
                rt = rb[b][pl.ds(i * 16, 16)]
                st = sb[b][pl.ds(i * 16, 16)]
                en = eb[b][pl.ds(i * 16, 16)]
                m = (st <= tv) & (tv < en)
                hm = idx < HALF
                n = jnp.where(hm, idx, idx - HALF)
                q = ((n.astype(jnp.uint32) * jnp.uint32(52429))
                     >> jnp.uint32(19)).astype(jnp.int32)
                off2 = n * PSTR + PSTR - q * (10 * PSTR)
                pos = q + jnp.where(hm, 0, off2)
                plsc.addupdate_scatter(acc, [pos >> 7, pos & 127], rt,
                                       mask=m)

        def outer(it, carry):
            g0 = it * NSLOT
            for b in range(NSLOT):
                g = g0 + b
                wait(b, g)

                @pl.when(g + NPRIME < NCH)
                def _():
                    start((b + NPRIME) % NSLOT, g + NPRIME)

                chunk_compute(b)
            return carry

        lax.fori_loop(0, (NCH - 1) // NSLOT, outer, 0)
        wait((NCH - 1) % NSLOT, NCH - 1)
        chunk_compute((NCH - 1) % NSLOT)

        plsc.subcore_barrier()
        for k in range(IDX_TILES):
            pltpu.sync_copy(acc.at[pl.ds(k * IDX_MINOR, IDX_MINOR)],
                            shacc.at[rows_i.at[k]], add=True)
        plsc.subcore_barrier()
        pltpu.sync_copy(shacc.at[pl.ds(sid * RPT, RPT)], wb)
        pltpu.sync_copy(wb, out_h.at[cid, pl.ds(sid * RPT, RPT)])

    return body(index, rate, starttime, endtime, tvec)


def _tc_combine(partials):
    PR = PSTR // ROWW

    def body(p_ref, out_ref):
        a = p_ref[0] + p_ref[1]
        s = a[0:PR, :].reshape(-1)
        m = a[PR:2 * PR, :]
        for w in range(1, POOLW):
            m = jnp.maximum(m, a[PR * (w + 1):PR * (w + 2), :])
        out_ref[pl.ds(0, NBUCK)] = s[:NBUCK]
        out_ref[pl.ds(NBUCK, NBUCK)] = m.reshape(-1)[:NBUCK]

    return pl.pallas_call(
        body,
        out_shape=jax.ShapeDtypeStruct((2 * NBUCK,), jnp.float32),
    )(partials)


@jax.jit
def kernel(index, rate, starttime, endtime, t):
    tvec = jnp.full((16,), t, jnp.float32)
    parts = _sc_partials(index.astype(jnp.int32), rate, starttime,
                         endtime, tvec)
    return _tc_combine(parts)
